# Initial kernel scaffold; baseline (speedup 1.0000x reference)
import jax
import jax.numpy as jnp
from jax.experimental import pallas as pl


def kernel(x, edge_index, edge_weight, batch, W_pre, b_pre, conv_W0, conv_b0, gn_w0, gn_b0, gn_s0, conv_W1, conv_b1, gn_w1, gn_b1, gn_s1, conv_W2, conv_b2, gn_w2, gn_b2, gn_s2, head_W0, head_b0, head_W1, head_b1, out_W, out_b):
    raise NotImplementedError("write your pallas kernel here")


# speedup vs baseline: 5.6651x; 5.6651x over previous
import functools

import jax
import jax.numpy as jnp
from jax import lax
from jax.experimental import pallas as pl
from jax.experimental.pallas import tpu as pltpu
from jax.experimental.pallas import tpu_sc as plsc

N = 10000
E = 320000
G = 64
H = 128
NEG = 0.1

NC = 2
NS = 16
NW = NC * NS
CH = 79
EPAD = NW * CH * 128
ACC = 10240
RPS = ACC // NS
DUMMY = N

@functools.lru_cache(maxsize=None)
def _sc_mesh():
    return plsc.VectorSubcoreMesh(
        core_axis_name="c", subcore_axis_name="s",
        num_cores=NC, num_subcores=NS)


def _lrelu(v):
    return jnp.where(v >= 0, v, NEG * v)


def _splat16(val_ref, i):
    return plsc.load_gather(val_ref, [jnp.full((16,), i, jnp.int32)])


@functools.lru_cache(maxsize=None)
def _deg_sc():
    return pl.kernel(
        _deg_sc_body,
        out_type=jax.ShapeDtypeStruct((NC, ACC, 128), jnp.float32),
        mesh=_sc_mesh(),
        compiler_params=pltpu.CompilerParams(needs_layout_passes=False),
        scratch_types=[
            pltpu.VMEM((CH, 128), jnp.int32),
            pltpu.VMEM((CH * 128,), jnp.float32),
            pltpu.VMEM((128, 128), jnp.float32),
            pltpu.VMEM_SHARED((ACC, 128), jnp.float32),
        ],
    )


def _deg_sc_body(dst_hbm, ew_hbm, out_hbm, dst_v, ew_v, rows, acc):
    c = lax.axis_index("c")
    s = lax.axis_index("s")
    pltpu.sync_copy(dst_hbm.at[c, s], dst_v)
    pltpu.sync_copy(ew_hbm.at[c, s], ew_v)

    zeros16 = jnp.zeros((16,), jnp.float32)

    def _zrow(i, _):
        for u in range(8):
            rows[i, pl.ds(u * 16, 16)] = zeros16
        return 0

    lax.fori_loop(0, 128, _zrow, 0)
    for k in range(RPS // 128):
        pltpu.sync_copy(rows, acc.at[pl.ds(s * RPS + k * 128, 128)])
    plsc.subcore_barrier()

    def _chunk(j, _):
        def _edge(e, _):
            w = _splat16(ew_v, j * 128 + e)
            for u in range(8):
                rows[e, pl.ds(u * 16, 16)] = w
            return 0

        lax.fori_loop(0, 128, _edge, 0)
        pltpu.sync_copy(rows, acc.at[dst_v.at[j]], add=True)
        return 0

    lax.fori_loop(0, CH, _chunk, 0)
    plsc.subcore_barrier()
    for k in range(RPS // 128):
        r0 = s * RPS + k * 128
        pltpu.sync_copy(acc.at[pl.ds(r0, 128)], out_hbm.at[c, pl.ds(r0, 128)])


@functools.lru_cache(maxsize=None)
def _conv_sc():
    return pl.kernel(
        _conv_sc_body,
        out_type=jax.ShapeDtypeStruct((NC, ACC, H), jnp.float32),
        mesh=_sc_mesh(),
        compiler_params=pltpu.CompilerParams(needs_layout_passes=False),
        scratch_types=[
            pltpu.VMEM((CH, 128), jnp.int32),
            pltpu.VMEM((CH, 128), jnp.int32),
            pltpu.VMEM((CH * 128,), jnp.float32),
            pltpu.VMEM((128, H), jnp.float32),
            pltpu.VMEM_SHARED((ACC, H), jnp.float32),
            pltpu.SemaphoreType.DMA,
        ],
    )


def _conv_sc_body(src_hbm, dst_hbm, ew_hbm, y_hbm, out_hbm,
                  src_v, dst_v, ew_v, rows, acc, sem):
    c = lax.axis_index("c")
    s = lax.axis_index("s")
    pltpu.sync_copy(src_hbm.at[c, s], src_v)
    pltpu.sync_copy(dst_hbm.at[c, s], dst_v)
    pltpu.sync_copy(ew_hbm.at[c, s], ew_v)

    zeros16 = jnp.zeros((16,), jnp.float32)

    def _zrow(i, _):
        for u in range(H // 16):
            rows[i, pl.ds(u * 16, 16)] = zeros16
        return 0

    lax.fori_loop(0, 128, _zrow, 0)
    for k in range(RPS // 128):
        pltpu.sync_copy(rows, acc.at[pl.ds(s * RPS + k * 128, 128)])
    plsc.subcore_barrier()

    def _chunk(j, _):
        pltpu.async_copy(y_hbm.at[src_v.at[j]], rows, sem).wait()

        def _edge(e, _):
            w = _splat16(ew_v, j * 128 + e)
            for u in range(H // 16):
                sl = pl.ds(u * 16, 16)
                rows[e, sl] = rows[e, sl] * w
            return 0

        lax.fori_loop(0, 128, _edge, 0)
        pltpu.sync_copy(rows, acc.at[dst_v.at[j]], add=True)
        return 0

    lax.fori_loop(0, CH, _chunk, 0)
    plsc.subcore_barrier()
    for k in range(RPS // 128):
        r0 = s * RPS + k * 128
        pltpu.sync_copy(acc.at[pl.ds(r0, 128)], out_hbm.at[c, pl.ds(r0, 128)])


def _dot(a, b):
    return jnp.dot(a, b, preferred_element_type=jnp.float32)


def _split3(v):
    p1 = v.astype(jnp.bfloat16).astype(jnp.float32)
    r = v - p1
    p2 = r.astype(jnp.bfloat16).astype(jnp.float32)
    p3 = (r - p2).astype(jnp.bfloat16).astype(jnp.float32)
    return p1, p2, p3


def _mdot(m, v):
    p1, p2, p3 = _split3(v)
    return _dot(m, p1) + _dot(m, p2) + _dot(m, p3)


def _mdot_t(m, v):
    dn = (((0,), (0,)), ((), ()))
    p1, p2, p3 = _split3(v)
    return (lax.dot_general(m, p1, dn, preferred_element_type=jnp.float32)
            + lax.dot_general(m, p2, dn, preferred_element_type=jnp.float32)
            + lax.dot_general(m, p3, dn, preferred_element_type=jnp.float32))


def _pre_tc(x_ref, wpre_ref, bpre_ref, w0_ref, degp_ref,
            dis_ref, y0_ref):
    deg = degp_ref[0, :N, 0:1] + degp_ref[1, :N, 0:1]
    dis = jnp.where(deg > 0, lax.rsqrt(jnp.where(deg > 0, deg, 1.0)), 0.0)
    dis_ref[...] = dis
    h0 = _dot(x_ref[...], wpre_ref[...]) + bpre_ref[...]
    y0_ref[...] = dis * _dot(h0, w0_ref[...])


def _layer_tc_body(has_next, parts_ref, dis_ref, batch_ref,
                   cb_ref, gw_ref, gb_ref, gs_ref, wnext_ref,
                   *out_refs):
    if has_next:
        y_ref, psum_ref, pmax_ref, cnt_ref = out_refs
    else:
        psum_ref, pmax_ref, cnt_ref = out_refs
    dis = dis_ref[...]
    conv = dis * (parts_ref[0, :N, :] + parts_ref[1, :N, :]) + cb_ref[...]

    batch = batch_ref[...]
    gids = lax.broadcasted_iota(jnp.int32, (G, N), 0)
    M = (batch == gids).astype(jnp.float32)
    cnt = jnp.maximum(jnp.sum(M, axis=1, keepdims=True), 1.0)
    cnt_ref[...] = cnt

    mean = _mdot(M, conv) / cnt
    out = conv - _mdot_t(M, mean) * gs_ref[...]
    var = _mdot(M, out * out) / cnt
    istd = lax.rsqrt(var + 1e-5)
    h = _lrelu(out * _mdot_t(M, istd) * gw_ref[...] + gb_ref[...])

    psum_ref[...] = _mdot(M, h)

    neg_inf = jnp.float32(-jnp.inf)
    batch_col = batch.reshape(N, 1)

    def _gmax(g, _):
        masked = jnp.where(batch_col == g, h, neg_inf)
        pmax_ref[pl.ds(g, 1), :] = jnp.max(masked, axis=0, keepdims=True)
        return 0

    lax.fori_loop(0, G, _gmax, 0)

    if has_next:
        y_ref[...] = dis * _dot(h, wnext_ref[...])


def _head_tc(psums_ref, pmaxs_ref, cnt_ref, w0_ref, b0_ref,
             w1_ref, b1_ref, ow_ref, ob_ref, out_ref):
    cnt = cnt_ref[...]
    w0 = w0_ref[...]
    z = jnp.zeros((G, H), jnp.float32)
    for l in range(3):
        psum = psums_ref[l]
        pmax = pmaxs_ref[l]
        pmax = jnp.where(pmax == -jnp.inf, 0.0, pmax)
        z = z + _dot(psum / cnt, w0[l * H:(l + 1) * H, :])
        z = z + _dot(pmax, w0[(3 + l) * H:(4 + l) * H, :])
        z = z + _dot(psum, w0[(6 + l) * H:(7 + l) * H, :])
    z = _lrelu(z + b0_ref[...])
    z = _lrelu(_dot(z, w1_ref[...]) + b1_ref[...])
    out_ref[...] = _dot(z, ow_ref[...]) + ob_ref[...]


def kernel(x, edge_index, edge_weight, batch, W_pre, b_pre,
           conv_W0, conv_b0, gn_w0, gn_b0, gn_s0,
           conv_W1, conv_b1, gn_w1, gn_b1, gn_s1,
           conv_W2, conv_b2, gn_w2, gn_b2, gn_s2,
           head_W0, head_b0, head_W1, head_b1, out_W, out_b):
    f32 = jnp.float32
    src = edge_index[0]
    dst = edge_index[1]
    pad = EPAD - E
    src_p = jnp.concatenate([src, jnp.zeros((pad,), jnp.int32)])
    src_p = src_p.reshape(NC, NS, CH, 128)
    dst_p = jnp.concatenate([dst, jnp.full((pad,), DUMMY, jnp.int32)])
    dst_p = dst_p.reshape(NC, NS, CH, 128)
    ew_p = jnp.concatenate([edge_weight, jnp.zeros((pad,), f32)])
    ew_p = ew_p.reshape(NC, NS, CH * 128)

    deg_parts = _deg_sc()(dst_p, ew_p)

    dis, y = pl.pallas_call(
        _pre_tc,
        out_shape=(jax.ShapeDtypeStruct((N, 1), f32),
                   jax.ShapeDtypeStruct((N, H), f32)),
    )(x, W_pre, b_pre.reshape(1, H), conv_W0, deg_parts)

    batch2 = batch.reshape(1, N)
    convs = [(conv_b0, gn_w0, gn_b0, gn_s0, conv_W1),
             (conv_b1, gn_w1, gn_b1, gn_s1, conv_W2),
             (conv_b2, gn_w2, gn_b2, gn_s2, None)]
    psums, pmaxs, cnt = [], [], None
    for l, (cb, gw, gb, gs, wnext) in enumerate(convs):
        parts = _conv_sc()(src_p, dst_p, ew_p, y)
        has_next = wnext is not None
        outs = [jax.ShapeDtypeStruct((G, H), f32)] * 3
        if has_next:
            outs = [jax.ShapeDtypeStruct((N, H), f32)] + outs
        outs[-1] = jax.ShapeDtypeStruct((G, 1), f32)
        res = pl.pallas_call(
            functools.partial(_layer_tc_body, has_next),
            out_shape=tuple(outs),
        )(parts, dis, batch2, cb.reshape(1, H), gw.reshape(1, H),
          gb.reshape(1, H), gs.reshape(1, H),
          wnext if has_next else jnp.zeros((1, 1), f32))
        if has_next:
            y, psum, pmax, cnt_l = res
        else:
            psum, pmax, cnt_l = res
        psums.append(psum)
        pmaxs.append(pmax)
        if cnt is None:
            cnt = cnt_l

    out = pl.pallas_call(
        _head_tc,
        out_shape=jax.ShapeDtypeStruct((G, 2), f32),
    )(jnp.stack(psums), jnp.stack(pmaxs), cnt, head_W0,
      head_b0.reshape(1, H), head_W1, head_b1.reshape(1, H),
      out_W, out_b.reshape(1, 2))
    return out
